# baseline (speedup 1.0000x reference)
import jax
import jax.numpy as jnp
from jax import lax
from jax.experimental import pallas as pl
from jax.experimental.pallas import tpu as pltpu
from jax.experimental.pallas import tpu_sc as plsc

N_NODES = 10000
N_PAD = 10240
N_EDGES = 320000
D = 128
NUM_GRAPHS = 64
CHUNK = 128
NCHUNKS = N_EDGES // CHUNK
NWORKERS = 32
KMAX = -(-NCHUNKS // NWORKERS)
NCHUNKS_P = 2560
KPW = NCHUNKS_P // NWORKERS
IBATCH = 8
ROWS_PER_SUB = N_PAD // 16
ROW_BLK = 1000
GRID = N_NODES // ROW_BLK

_SC_MESH = plsc.VectorSubcoreMesh(core_axis_name="c", subcore_axis_name="s")


def _zero_acc(const_hbm, acc_sh, sid):
    zrow = sid * ROWS_PER_SUB
    for j in range(ROWS_PER_SUB // CHUNK):
        pltpu.sync_copy(const_hbm, acc_sh.at[pl.ds(zrow + j * CHUNK, CHUNK)])


def _writeout(acc_sh, out0, out1, cid, sid, bounce):
    zrow = sid * ROWS_PER_SUB
    for j in range(ROWS_PER_SUB // CHUNK):
        off = zrow + j * CHUNK
        pltpu.sync_copy(acc_sh.at[pl.ds(off, CHUNK)], bounce)

        @pl.when(cid == 0)
        def _():
            pltpu.sync_copy(bounce, out0.at[pl.ds(off, CHUNK)])

        @pl.when(cid == 1)
        def _():
            pltpu.sync_copy(bounce, out1.at[pl.ds(off, CHUNK)])


def _sc_scatter_body(idx_hbm, g_hbm, const_hbm, out0, out1,
                     ib, rows_v, acc_sh, sem):
    cid = lax.axis_index("c")
    sid = lax.axis_index("s")
    wid = sid * 2 + cid
    base = wid * KPW

    _zero_acc(const_hbm, acc_sh, sid)
    plsc.subcore_barrier()

    @pl.loop(0, KPW // IBATCH)
    def _(i):
        pltpu.sync_copy(idx_hbm.at[pl.ds(base + i * IBATCH, IBATCH)], ib)
        for s in range(IBATCH):
            pltpu.async_copy(g_hbm.at[ib.at[s, 0]], rows_v, sem).wait()
            pltpu.sync_copy(rows_v, acc_sh.at[ib.at[s, 1]], add=True)

    plsc.subcore_barrier()
    _writeout(acc_sh, out0, out1, cid, sid, rows_v)


_sc_scatter = pl.kernel(
    _sc_scatter_body,
    out_type=(jax.ShapeDtypeStruct((N_PAD, D), jnp.float32),
              jax.ShapeDtypeStruct((N_PAD, D), jnp.float32)),
    mesh=_SC_MESH,
    scratch_types=[
        pltpu.VMEM((IBATCH, 2, CHUNK), jnp.int32),
        pltpu.VMEM((CHUNK, D), jnp.float32),
        pltpu.VMEM_SHARED((N_PAD, D), jnp.float32),
        pltpu.SemaphoreType.DMA,
    ],
)


def _sc_degree_body(dst_hbm, ones_hbm, const_hbm, out0, out1,
                    dst_v, rows_v, acc_sh):
    cid = lax.axis_index("c")
    sid = lax.axis_index("s")
    wid = sid * 2 + cid

    _zero_acc(const_hbm, acc_sh, sid)
    pltpu.sync_copy(ones_hbm, rows_v)
    plsc.subcore_barrier()

    @pl.loop(0, KMAX)
    def _(k):
        ch = k * NWORKERS + wid

        @pl.when(ch < NCHUNKS)
        def _():
            base = ch * CHUNK
            pltpu.sync_copy(dst_hbm.at[pl.ds(base, CHUNK)], dst_v)
            pltpu.sync_copy(rows_v, acc_sh.at[dst_v], add=True)

    plsc.subcore_barrier()
    _writeout(acc_sh, out0, out1, cid, sid, rows_v)


_sc_degree = pl.kernel(
    _sc_degree_body,
    out_type=(jax.ShapeDtypeStruct((N_PAD, D), jnp.float32),
              jax.ShapeDtypeStruct((N_PAD, D), jnp.float32)),
    mesh=_SC_MESH,
    scratch_types=[
        pltpu.VMEM((CHUNK,), jnp.int32),
        pltpu.VMEM((CHUNK, D), jnp.float32),
        pltpu.VMEM_SHARED((N_PAD, D), jnp.float32),
    ],
)


def _dinv_body(d0_ref, d1_ref, out_ref):
    deg = d0_ref[...] + d1_ref[...] + 1.0
    out_ref[...] = lax.rsqrt(deg)


def _dinv_kernel(d0, d1):
    blk = pl.BlockSpec((1024, D), lambda i: (i, 0))
    return pl.pallas_call(
        _dinv_body,
        grid=(N_PAD // 1024,),
        in_specs=[blk, blk],
        out_specs=blk,
        out_shape=jax.ShapeDtypeStruct((N_PAD, D), jnp.float32),
    )(d0, d1)


def _dot(a, b):
    return jax.lax.dot_general(a, b, (((1,), (0,)), ((), ())),
                               precision=lax.Precision.HIGHEST,
                               preferred_element_type=jnp.float32)


def _prep1_body(x_ref, w0_ref, b0_ref, w1_ref, dinv_ref, out_ref):
    h = jnp.maximum(_dot(x_ref[...], w0_ref[...]) + b0_ref[...][None, :], 0.0)
    out_ref[...] = dinv_ref[...] * _dot(h, w1_ref[...])


def _prep1(x, W0, b0, Wc1, dinv):
    rblk = pl.BlockSpec((ROW_BLK, D), lambda i: (i, 0))
    wblk = pl.BlockSpec((D, D), lambda i: (0, 0))
    bblk = pl.BlockSpec((D,), lambda i: (0,))
    return pl.pallas_call(
        _prep1_body,
        grid=(GRID,),
        in_specs=[rblk, wblk, bblk, wblk, rblk],
        out_specs=rblk,
        out_shape=jax.ShapeDtypeStruct((N_NODES, D), jnp.float32),
    )(x, W0, b0, Wc1, dinv)


def _comb_body(p0_ref, p1_ref, g_ref, dinv_ref, b_ref, w_ref, out_ref):
    dinv = dinv_ref[...]
    h = dinv * (p0_ref[...] + p1_ref[...] + g_ref[...]) + b_ref[...][None, :]
    h = jnp.maximum(h, 0.0)
    out_ref[...] = dinv * _dot(h, w_ref[...])


def _comb(p0, p1, g, dinv, b, W):
    rblk = pl.BlockSpec((ROW_BLK, D), lambda i: (i, 0))
    wblk = pl.BlockSpec((D, D), lambda i: (0, 0))
    bblk = pl.BlockSpec((D,), lambda i: (0,))
    return pl.pallas_call(
        _comb_body,
        grid=(GRID,),
        in_specs=[rblk, rblk, rblk, rblk, bblk, wblk],
        out_specs=rblk,
        out_shape=jax.ShapeDtypeStruct((N_NODES, D), jnp.float32),
    )(p0, p1, g, dinv, b, W)


def _final_body(p0_ref, p1_ref, g_ref, dinv_ref, b_ref, batch_ref, out_ref,
                sums_ref, cnts_ref):
    i = pl.program_id(0)

    @pl.when(i == 0)
    def _():
        sums_ref[...] = jnp.zeros_like(sums_ref)
        cnts_ref[...] = jnp.zeros_like(cnts_ref)

    dinv = dinv_ref[...]
    h = dinv * (p0_ref[...] + p1_ref[...] + g_ref[...]) + b_ref[...][None, :]
    h = jnp.maximum(h, 0.0)
    seg = batch_ref[0, 0, :]
    gids = lax.broadcasted_iota(jnp.int32, (NUM_GRAPHS, ROW_BLK), 0)
    onehot = jnp.where(gids == seg[None, :], 1.0, 0.0)
    sums_ref[...] += _dot(onehot, h)
    cnt = jnp.sum(onehot, axis=1, keepdims=True)
    cnts_ref[...] += jnp.broadcast_to(cnt, (NUM_GRAPHS, D))

    @pl.when(i == GRID - 1)
    def _():
        out_ref[...] = sums_ref[...] / jnp.maximum(cnts_ref[...], 1.0)


def _final_pool(p0, p1, g, dinv, b, batch3):
    rblk = pl.BlockSpec((ROW_BLK, D), lambda i: (i, 0))
    bblk = pl.BlockSpec((D,), lambda i: (0,))
    sblk = pl.BlockSpec((1, 1, ROW_BLK), lambda i: (i, 0, 0))
    oblk = pl.BlockSpec((NUM_GRAPHS, D), lambda i: (0, 0))
    return pl.pallas_call(
        _final_body,
        grid=(GRID,),
        in_specs=[rblk, rblk, rblk, rblk, bblk, sblk],
        out_specs=oblk,
        out_shape=jax.ShapeDtypeStruct((NUM_GRAPHS, D), jnp.float32),
        scratch_shapes=[pltpu.VMEM((NUM_GRAPHS, D), jnp.float32),
                        pltpu.VMEM((NUM_GRAPHS, D), jnp.float32)],
    )(p0, p1, g, dinv, b, batch3)


def kernel(x, edge_index, batch, W0, b0, Wc1, bc1, Wc2, bc2, Wc3, bc3):
    src = edge_index[0]
    dst = edge_index[1]
    pad = NCHUNKS_P * CHUNK - N_EDGES
    src_p = jnp.concatenate([src, jnp.zeros((pad,), jnp.int32)])
    dst_p = jnp.concatenate([dst, jnp.full((pad,), N_NODES, jnp.int32)])
    idx = jnp.stack([src_p.reshape(NCHUNKS_P, CHUNK),
                     dst_p.reshape(NCHUNKS_P, CHUNK)], axis=1)
    zeros_blk = jnp.zeros((CHUNK, D), jnp.float32)
    ones_blk = jnp.ones((CHUNK, D), jnp.float32)

    d0, d1 = _sc_degree(dst, ones_blk, zeros_blk)
    dinv = _dinv_kernel(d0, d1)

    g = _prep1(x, W0, b0, Wc1, dinv)
    p0, p1 = _sc_scatter(idx, g, zeros_blk)
    g = _comb(p0[:N_NODES], p1[:N_NODES], g, dinv[:N_NODES], bc1, Wc2)
    p0, p1 = _sc_scatter(idx, g, zeros_blk)
    g = _comb(p0[:N_NODES], p1[:N_NODES], g, dinv[:N_NODES], bc2, Wc3)
    p0, p1 = _sc_scatter(idx, g, zeros_blk)

    batch3 = batch.reshape(GRID, 1, ROW_BLK)
    return _final_pool(p0[:N_NODES], p1[:N_NODES], g, dinv[:N_NODES], bc3,
                       batch3)


# speedup vs baseline: 2.3038x; 2.3038x over previous
import jax
import jax.numpy as jnp
from jax import lax
from jax.experimental import pallas as pl
from jax.experimental.pallas import tpu as pltpu
from jax.experimental.pallas import tpu_sc as plsc

N_NODES = 10000
N_PAD = 10240
N_EDGES = 320000
D = 128
NUM_GRAPHS = 64
CHUNK = 128
NCHUNKS = N_EDGES // CHUNK
NWORKERS = 32
KMAX = -(-NCHUNKS // NWORKERS)
NCHUNKS_P = 2560
KPW = NCHUNKS_P // NWORKERS
IBATCH = 8
ROWS_PER_SUB = N_PAD // 16
ROW_BLK = 1000
GRID = N_NODES // ROW_BLK

_SC_MESH = plsc.VectorSubcoreMesh(core_axis_name="c", subcore_axis_name="s")


def _zero_acc(const_hbm, acc_sh, sid):
    zrow = sid * ROWS_PER_SUB
    for j in range(ROWS_PER_SUB // CHUNK):
        pltpu.sync_copy(const_hbm, acc_sh.at[pl.ds(zrow + j * CHUNK, CHUNK)])


def _writeout(acc_sh, out0, out1, cid, sid, bounce):
    zrow = sid * ROWS_PER_SUB
    for j in range(ROWS_PER_SUB // CHUNK):
        off = zrow + j * CHUNK
        pltpu.sync_copy(acc_sh.at[pl.ds(off, CHUNK)], bounce)

        @pl.when(cid == 0)
        def _():
            pltpu.sync_copy(bounce, out0.at[pl.ds(off, CHUNK)])

        @pl.when(cid == 1)
        def _():
            pltpu.sync_copy(bounce, out1.at[pl.ds(off, CHUNK)])


def _sc_scatter_body(idx_hbm, g_hbm, const_hbm, out0, out1,
                     ib, rows_v, acc_sh, sem):
    cid = lax.axis_index("c")
    sid = lax.axis_index("s")
    wid = sid * 2 + cid
    base = wid * KPW

    _zero_acc(const_hbm, acc_sh, sid)
    plsc.subcore_barrier()

    @pl.loop(0, KPW // IBATCH)
    def _(i):
        pltpu.sync_copy(idx_hbm.at[pl.ds(base + i * IBATCH, IBATCH)], ib)
        for s in range(IBATCH):
            pltpu.async_copy(g_hbm.at[ib.at[s, 0]], rows_v, sem).wait()
            pltpu.sync_copy(rows_v, acc_sh.at[ib.at[s, 1]], add=True)

    plsc.subcore_barrier()
    _writeout(acc_sh, out0, out1, cid, sid, rows_v)


_sc_scatter = pl.kernel(
    _sc_scatter_body,
    out_type=(jax.ShapeDtypeStruct((N_PAD, D), jnp.float32),
              jax.ShapeDtypeStruct((N_PAD, D), jnp.float32)),
    mesh=_SC_MESH,
    scratch_types=[
        pltpu.VMEM((IBATCH, 2, CHUNK), jnp.int32),
        pltpu.VMEM((CHUNK, D), jnp.float32),
        pltpu.VMEM_SHARED((N_PAD, D), jnp.float32),
        pltpu.SemaphoreType.DMA,
    ],
)


def _sc_degree_body(dst_hbm, ones_hbm, const_hbm, out0, out1,
                    dst_v, rows_v, acc_sh):
    cid = lax.axis_index("c")
    sid = lax.axis_index("s")
    wid = sid * 2 + cid

    _zero_acc(const_hbm, acc_sh, sid)
    pltpu.sync_copy(ones_hbm, rows_v)
    plsc.subcore_barrier()

    @pl.loop(0, KMAX)
    def _(k):
        ch = k * NWORKERS + wid

        @pl.when(ch < NCHUNKS)
        def _():
            base = ch * CHUNK
            pltpu.sync_copy(dst_hbm.at[pl.ds(base, CHUNK)], dst_v)
            pltpu.sync_copy(rows_v, acc_sh.at[dst_v], add=True)

    plsc.subcore_barrier()
    _writeout(acc_sh, out0, out1, cid, sid, rows_v)


_sc_degree = pl.kernel(
    _sc_degree_body,
    out_type=(jax.ShapeDtypeStruct((N_PAD, D), jnp.float32),
              jax.ShapeDtypeStruct((N_PAD, D), jnp.float32)),
    mesh=_SC_MESH,
    scratch_types=[
        pltpu.VMEM((CHUNK,), jnp.int32),
        pltpu.VMEM((CHUNK, D), jnp.float32),
        pltpu.VMEM_SHARED((N_PAD, D), jnp.float32),
    ],
)


def _dinv_body(d0_ref, d1_ref, out_ref):
    deg = d0_ref[...] + d1_ref[...] + 1.0
    out_ref[...] = lax.rsqrt(deg)


def _dinv_kernel(d0, d1):
    blk = pl.BlockSpec((1024, D), lambda i: (i, 0))
    return pl.pallas_call(
        _dinv_body,
        grid=(N_PAD // 1024,),
        in_specs=[blk, blk],
        out_specs=blk,
        out_shape=jax.ShapeDtypeStruct((N_PAD, D), jnp.float32),
    )(d0, d1)


def _dot(a, b):
    return jax.lax.dot_general(a, b, (((1,), (0,)), ((), ())),
                               precision=lax.Precision.HIGHEST,
                               preferred_element_type=jnp.float32)


def _prep1_body(x_ref, w0_ref, b0_ref, w1_ref, dinv_ref, out_ref):
    h = jnp.maximum(_dot(x_ref[...], w0_ref[...]) + b0_ref[...][None, :], 0.0)
    out_ref[...] = dinv_ref[...] * _dot(h, w1_ref[...])


def _prep1(x, W0, b0, Wc1, dinv):
    rblk = pl.BlockSpec((ROW_BLK, D), lambda i: (i, 0))
    wblk = pl.BlockSpec((D, D), lambda i: (0, 0))
    bblk = pl.BlockSpec((D,), lambda i: (0,))
    return pl.pallas_call(
        _prep1_body,
        grid=(GRID,),
        in_specs=[rblk, wblk, bblk, wblk, rblk],
        out_specs=rblk,
        out_shape=jax.ShapeDtypeStruct((N_NODES, D), jnp.float32),
    )(x, W0, b0, Wc1, dinv)


def _comb_body(p0_ref, p1_ref, g_ref, dinv_ref, b_ref, w_ref, out_ref):
    dinv = dinv_ref[...]
    h = dinv * (p0_ref[...] + p1_ref[...] + g_ref[...]) + b_ref[...][None, :]
    h = jnp.maximum(h, 0.0)
    out_ref[...] = dinv * _dot(h, w_ref[...])


def _comb(p0, p1, g, dinv, b, W):
    rblk = pl.BlockSpec((ROW_BLK, D), lambda i: (i, 0))
    wblk = pl.BlockSpec((D, D), lambda i: (0, 0))
    bblk = pl.BlockSpec((D,), lambda i: (0,))
    return pl.pallas_call(
        _comb_body,
        grid=(GRID,),
        in_specs=[rblk, rblk, rblk, rblk, bblk, wblk],
        out_specs=rblk,
        out_shape=jax.ShapeDtypeStruct((N_NODES, D), jnp.float32),
    )(p0, p1, g, dinv, b, W)


def _final_body(p0_ref, p1_ref, g_ref, dinv_ref, b_ref, batch_ref, out_ref,
                sums_ref, cnts_ref):
    i = pl.program_id(0)

    @pl.when(i == 0)
    def _():
        sums_ref[...] = jnp.zeros_like(sums_ref)
        cnts_ref[...] = jnp.zeros_like(cnts_ref)

    dinv = dinv_ref[...]
    h = dinv * (p0_ref[...] + p1_ref[...] + g_ref[...]) + b_ref[...][None, :]
    h = jnp.maximum(h, 0.0)
    seg = batch_ref[0, 0, :]
    gids = lax.broadcasted_iota(jnp.int32, (NUM_GRAPHS, ROW_BLK), 0)
    onehot = jnp.where(gids == seg[None, :], 1.0, 0.0)
    sums_ref[...] += _dot(onehot, h)
    cnt = jnp.sum(onehot, axis=1, keepdims=True)
    cnts_ref[...] += jnp.broadcast_to(cnt, (NUM_GRAPHS, D))

    @pl.when(i == GRID - 1)
    def _():
        out_ref[...] = sums_ref[...] / jnp.maximum(cnts_ref[...], 1.0)


def _final_pool(p0, p1, g, dinv, b, batch3):
    rblk = pl.BlockSpec((ROW_BLK, D), lambda i: (i, 0))
    bblk = pl.BlockSpec((D,), lambda i: (0,))
    sblk = pl.BlockSpec((1, 1, ROW_BLK), lambda i: (i, 0, 0))
    oblk = pl.BlockSpec((NUM_GRAPHS, D), lambda i: (0, 0))
    return pl.pallas_call(
        _final_body,
        grid=(GRID,),
        in_specs=[rblk, rblk, rblk, rblk, bblk, sblk],
        out_specs=oblk,
        out_shape=jax.ShapeDtypeStruct((NUM_GRAPHS, D), jnp.float32),
        scratch_shapes=[pltpu.VMEM((NUM_GRAPHS, D), jnp.float32),
                        pltpu.VMEM((NUM_GRAPHS, D), jnp.float32)],
    )(p0, p1, g, dinv, b, batch3)


def kernel(x, edge_index, batch, W0, b0, Wc1, bc1, Wc2, bc2, Wc3, bc3):
    src = edge_index[0]
    dst = edge_index[1]
    pad = NCHUNKS_P * CHUNK - N_EDGES
    spread = jnp.arange(pad, dtype=jnp.int32)
    src_p = jnp.concatenate([src, spread % N_NODES])
    dst_p = jnp.concatenate(
        [dst, N_NODES + (spread % (N_PAD - N_NODES))])
    idx = jnp.stack([src_p.reshape(NCHUNKS_P, CHUNK),
                     dst_p.reshape(NCHUNKS_P, CHUNK)], axis=1)
    zeros_blk = jnp.zeros((CHUNK, D), jnp.float32)
    ones_blk = jnp.ones((CHUNK, D), jnp.float32)

    d0, d1 = _sc_degree(dst, ones_blk, zeros_blk)
    dinv = _dinv_kernel(d0, d1)

    g = _prep1(x, W0, b0, Wc1, dinv)
    p0, p1 = _sc_scatter(idx, g, zeros_blk)
    g = _comb(p0[:N_NODES], p1[:N_NODES], g, dinv[:N_NODES], bc1, Wc2)
    p0, p1 = _sc_scatter(idx, g, zeros_blk)
    g = _comb(p0[:N_NODES], p1[:N_NODES], g, dinv[:N_NODES], bc2, Wc3)
    p0, p1 = _sc_scatter(idx, g, zeros_blk)

    batch3 = batch.reshape(GRID, 1, ROW_BLK)
    return _final_pool(p0[:N_NODES], p1[:N_NODES], g, dinv[:N_NODES], bc3,
                       batch3)
